# baseline (speedup 1.0000x reference)
import functools

import jax
import jax.numpy as jnp
from jax import lax
from jax.experimental import pallas as pl
from jax.experimental.pallas import tpu as pltpu
from jax.experimental.pallas import tpu_sc as plsc

FIELD = 38461
NFIELD = 26
EMBED = 16
BATCH = 16384
NROWS = FIELD * NFIELD
NC, NS, LANES = 2, 16, 16
F_PER_CORE = NFIELD // NC
WIN = 38656
TAIL_COL = (NROWS // 128) * 128
TAIL_WIN = 38528
TAIL_PAD = 128
NVEC = BATCH // LANES


def _body(x_hbm, tab_hbm, aux_hbm, out_hbm, xrow_a, xrow_b, win_a,
          win_b, out_v, sem_w0, sem_w1, sem_x0, sem_x1, sem_o):
    c = lax.axis_index("s")
    k = lax.axis_index("c")
    xrows = (xrow_a, xrow_b)
    wins = (win_a, win_b)
    sems_w = (sem_w0, sem_w1)
    sems_x = (sem_x0, sem_x1)

    def load_xrow(j):
        b = j % 2
        return pltpu.async_copy(x_hbm.at[k * F_PER_CORE + j, :],
                                xrows[b], sems_x[b])

    def load_win(j):
        b = j % 2
        f = k * F_PER_CORE + j
        o = f * FIELD
        col0 = pl.multiple_of(o - lax.rem(o, 128), 128)
        descs = []
        win = wins[b]
        if j == F_PER_CORE - 1:
            @pl.when(k == 0)
            def _full():
                pltpu.async_copy(tab_hbm.at[c, pl.ds(col0, WIN)],
                                 win.at[pl.ds(0, WIN)], sems_w[b])

            @pl.when(k == 1)
            def _tail():
                pltpu.async_copy(tab_hbm.at[c, pl.ds(col0, TAIL_WIN)],
                                 win.at[pl.ds(0, TAIL_WIN)], sems_w[b])
                pltpu.async_copy(aux_hbm.at[c, :],
                                 win.at[pl.ds(TAIL_WIN, TAIL_PAD)], sems_w[b])
            descs.append(pltpu.make_async_copy(
                tab_hbm.at[c, pl.ds(0, WIN)], win, sems_w[b]))
        else:
            descs.append(pltpu.async_copy(tab_hbm.at[c, pl.ds(col0, WIN)],
                                          win, sems_w[b]))
        return descs

    out_desc = None
    descs = load_win(0)
    xds = [load_xrow(0), None]
    for j in range(F_PER_CORE):
        b = j % 2
        f = k * F_PER_CORE + j
        o = f * FIELD
        pad = o - (o - lax.rem(o, 128))
        for d in descs:
            d.wait()
        if j + 1 < F_PER_CORE:
            descs = load_win(j + 1)
            xds[(j + 1) % 2] = load_xrow(j + 1)
        xds[b].wait()
        if out_desc is not None:
            out_desc.wait()

        @plsc.parallel_loop(0, BATCH, step=LANES, unroll=4)
        def _gather(g, b=b, pad=pad):
            idx = xrows[b][pl.ds(g, LANES)] + pad
            out_v[pl.ds(g, LANES)] = plsc.load_gather(wins[b], [idx])

        out_desc = pltpu.async_copy(out_v, out_hbm.at[f, c, :], sem_o)
    out_desc.wait()


def kernel(x, table):
    x_t = x.T
    tab_t = table.T
    aux = jnp.pad(table[TAIL_COL:, :].T, ((0, 0), (0, TAIL_PAD - (NROWS - TAIL_COL))))
    mesh = plsc.VectorSubcoreMesh(core_axis_name="c", subcore_axis_name="s")
    run = functools.partial(
        pl.kernel,
        mesh=mesh,
        out_type=jax.ShapeDtypeStruct((NFIELD, EMBED, BATCH), jnp.float32),
        scratch_types=[
            pltpu.VMEM((BATCH,), jnp.int32),
            pltpu.VMEM((BATCH,), jnp.int32),
            pltpu.VMEM((WIN,), jnp.float32),
            pltpu.VMEM((WIN,), jnp.float32),
            pltpu.VMEM((BATCH,), jnp.float32),
            pltpu.SemaphoreType.DMA,
            pltpu.SemaphoreType.DMA,
            pltpu.SemaphoreType.DMA,
            pltpu.SemaphoreType.DMA,
            pltpu.SemaphoreType.DMA,
        ],
        compiler_params=pltpu.CompilerParams(needs_layout_passes=False),
    )(_body)
    out = run(x_t, tab_t, aux)
    return out.transpose(2, 0, 1)


# speedup vs baseline: 1.3440x; 1.3440x over previous
import functools

import jax
import jax.numpy as jnp
from jax import lax
from jax.experimental import pallas as pl
from jax.experimental.pallas import tpu as pltpu
from jax.experimental.pallas import tpu_sc as plsc

FIELD = 38461
NFIELD = 26
EMBED = 16
BATCH = 16384
NROWS = FIELD * NFIELD
NC, NS, LANES = 2, 16, 16
F_PER_CORE = NFIELD // NC
WIN = 38656
TAIL_COL = (NROWS // 128) * 128
TAIL_WIN = 38528
TAIL_PAD = 128


def _body(x_hbm, tab_hbm, aux_hbm, out_hbm, xsh_v, xrow_v, win_a,
          win_b, out_v, sem_w0, sem_w1, sem_x, sem_o, sem_s):
    c = lax.axis_index("s")
    k = lax.axis_index("c")
    wins = (win_a, win_b)
    sems_w = (sem_w0, sem_w1)

    def load_xrow(j):
        return pltpu.async_copy(xsh_v.at[pl.ds(j * BATCH, BATCH)],
                                xrow_v, sem_x)

    def load_win(j):
        b = j % 2
        f = k * F_PER_CORE + j
        o = f * FIELD
        col0 = pl.multiple_of(o - lax.rem(o, 128), 128)
        descs = []
        win = wins[b]
        if j == F_PER_CORE - 1:
            @pl.when(k == 0)
            def _full():
                pltpu.async_copy(tab_hbm.at[c, pl.ds(col0, WIN)],
                                 win.at[pl.ds(0, WIN)], sems_w[b])

            @pl.when(k == 1)
            def _tail():
                pltpu.async_copy(tab_hbm.at[c, pl.ds(col0, TAIL_WIN)],
                                 win.at[pl.ds(0, TAIL_WIN)], sems_w[b])
                pltpu.async_copy(aux_hbm.at[c, :],
                                 win.at[pl.ds(TAIL_WIN, TAIL_PAD)], sems_w[b])
            descs.append(pltpu.make_async_copy(
                tab_hbm.at[c, pl.ds(0, WIN)], win, sems_w[b]))
        else:
            descs.append(pltpu.async_copy(tab_hbm.at[c, pl.ds(col0, WIN)],
                                          win, sems_w[b]))
        return descs

    descs = load_win(0)

    @pl.when(c == 0)
    def _stage_x():
        stage = [
            pltpu.async_copy(x_hbm.at[k * F_PER_CORE + jj, :],
                             xsh_v.at[pl.ds(jj * BATCH, BATCH)], sem_s)
            for jj in range(F_PER_CORE)
        ]
        for d in stage:
            d.wait()

    plsc.subcore_barrier()

    out_desc = None
    xd = load_xrow(0)
    for j in range(F_PER_CORE):
        b = j % 2
        f = k * F_PER_CORE + j
        o = f * FIELD
        pad = o - (o - lax.rem(o, 128))
        for d in descs:
            d.wait()
        if j + 1 < F_PER_CORE:
            descs = load_win(j + 1)
        xd.wait()
        if out_desc is not None:
            out_desc.wait()

        @plsc.parallel_loop(0, BATCH, step=LANES, unroll=4)
        def _gather(g, b=b, pad=pad):
            idx = xrow_v[pl.ds(g, LANES)] + pad
            out_v[pl.ds(g, LANES)] = plsc.load_gather(wins[b], [idx])

        if j + 1 < F_PER_CORE:
            xd = load_xrow(j + 1)
        out_desc = pltpu.async_copy(out_v, out_hbm.at[f, c, :], sem_o)
    out_desc.wait()


def kernel(x, table):
    x_t = x.T
    tab_t = table.T
    aux = jnp.pad(table[TAIL_COL:, :].T,
                  ((0, 0), (0, TAIL_PAD - (NROWS - TAIL_COL))))
    mesh = plsc.VectorSubcoreMesh(core_axis_name="c", subcore_axis_name="s")
    run = functools.partial(
        pl.kernel,
        mesh=mesh,
        out_type=jax.ShapeDtypeStruct((NFIELD, EMBED, BATCH), jnp.float32),
        scratch_types=[
            pltpu.VMEM_SHARED((F_PER_CORE * BATCH,), jnp.int32),
            pltpu.VMEM((BATCH,), jnp.int32),
            pltpu.VMEM((WIN,), jnp.float32),
            pltpu.VMEM((WIN,), jnp.float32),
            pltpu.VMEM((BATCH,), jnp.float32),
            pltpu.SemaphoreType.DMA,
            pltpu.SemaphoreType.DMA,
            pltpu.SemaphoreType.DMA,
            pltpu.SemaphoreType.DMA,
            pltpu.SemaphoreType.DMA,
        ],
        compiler_params=pltpu.CompilerParams(needs_layout_passes=False),
    )(_body)
    out = run(x_t, tab_t, aux)
    return out.transpose(2, 0, 1)
